# baseline (speedup 1.0000x reference)
import jax
import jax.numpy as jnp
from jax import lax
from jax.experimental import pallas as pl
from jax.experimental.pallas import tpu as pltpu
from jax.experimental.pallas import tpu_sc as plsc

_N = 1000000
_FLAT = 3 * _N
_NC = 2
_NS = 16
_NW = _NC * _NS
_LANES = 16
_TILE_ROWS = 768
_ROWS = _NW * _TILE_ROWS
_PAD_FLAT = _ROWS * 128
_NCHUNK = 4
_CHUNK_ROWS = _TILE_ROWS // _NCHUNK
_QBLOCKS = _CHUNK_ROWS // 3
_STEPS = [128, 64, 32, 16, 8, 4, 2, 1]


def _qbody(vals_hbm, table_hbm, out_hbm, table_v, in_v, out_v):
    wid = lax.axis_index("s") * _NC + lax.axis_index("c")
    row_base = wid * _TILE_ROWS
    pltpu.sync_copy(table_hbm, table_v)
    iota = lax.iota(jnp.int32, _LANES)
    i0 = [(lax.rem(iota + p, 3) << 8) + 127 for p in range(3)]

    for c in range(_NCHUNK):
        rstart = row_base + c * _CHUNK_ROWS
        pltpu.sync_copy(vals_hbm.at[pl.ds(rstart, _CHUNK_ROWS), :], in_v)

        @plsc.parallel_loop(0, _QBLOCKS, 1, unroll=2)
        def vbody(q):
            row0 = q * 3
            for t in range(3):
                for j in range(8):
                    x = in_v[row0 + t, pl.ds(j * _LANES, _LANES)]
                    i = i0[(2 * t + j) % 3]
                    for k, s in enumerate(_STEPS):
                        b = plsc.load_gather(table_v, [i])
                        m = b <= x
                        s_next = _STEPS[k + 1] if k + 1 < len(_STEPS) else 1
                        i = i + jnp.where(m, s_next, s_next - s)
                    out_v[row0 + t, pl.ds(j * _LANES, _LANES)] = i & 255

        pltpu.sync_copy(out_v, out_hbm.at[pl.ds(rstart, _CHUNK_ROWS), :])


def kernel(values, arousal_bins, dominance_bins, valence_bins):
    flat = jnp.pad(jnp.reshape(values, (-1,)), (0, _PAD_FLAT - _FLAT))
    vals2d = flat.reshape(_ROWS, 128)
    table = jnp.concatenate([arousal_bins, dominance_bins, valence_bins])
    vals2d, table = lax.optimization_barrier((vals2d, table))
    run = pl.kernel(
        _qbody,
        out_type=jax.ShapeDtypeStruct((_ROWS, 128), jnp.int32),
        mesh=plsc.VectorSubcoreMesh(core_axis_name="c", subcore_axis_name="s"),
        compiler_params=pltpu.CompilerParams(needs_layout_passes=False),
        scratch_types=[
            pltpu.VMEM((3 * 256,), jnp.float32),
            pltpu.VMEM((_CHUNK_ROWS, 128), jnp.float32),
            pltpu.VMEM((_CHUNK_ROWS, 128), jnp.int32),
        ],
    )
    out = lax.optimization_barrier(run(vals2d, table))
    return out.reshape(-1)[:_FLAT].reshape(_N, 3)


# speedup vs baseline: 17.2607x; 17.2607x over previous
import jax
import jax.numpy as jnp
from jax import lax
from jax.experimental import pallas as pl
from jax.experimental.pallas import tpu as pltpu
from jax.experimental.pallas import tpu_sc as plsc

_N = 1000000
_NC = 2
_NS = 16
_NW = _NC * _NS
_LANES = 16
_PER = 31264
_PER_LAST = _N - (_NW - 1) * _PER
_NCHUNK = 2
_CH = _PER // _NCHUNK
_CH_LAST = _PER_LAST // _NCHUNK
_STEPS = [128, 64, 32, 16, 8, 4, 2, 1]


def _qbody(vt_hbm, table_hbm, out_hbm, table_v, in_v, out_v):
    wid = lax.axis_index("s") * _NC + lax.axis_index("c")
    pltpu.sync_copy(table_hbm, table_v)

    def do_range(base, ch):
        nvec = ch // _LANES
        for c in range(_NCHUNK):
            s = base + c * ch
            for j in range(3):
                pltpu.sync_copy(
                    vt_hbm.at[j, pl.ds(s, ch)], in_v.at[j, pl.ds(0, ch)]
                )

            @plsc.parallel_loop(0, nvec, 1, unroll=4)
            def vbody(g):
                off = g * _LANES
                for j in range(3):
                    x = in_v[j, pl.ds(off, _LANES)]
                    i = jnp.broadcast_to(jnp.int32(j * 256 + 127), (_LANES,))
                    for k, st in enumerate(_STEPS):
                        b = plsc.load_gather(table_v, [i])
                        m = b <= x
                        s_next = _STEPS[k + 1] if k + 1 < len(_STEPS) else 1
                        i = i + jnp.where(m, s_next, s_next - st)
                    out_v[j, pl.ds(off, _LANES)] = i & 255

            for j in range(3):
                pltpu.sync_copy(
                    out_v.at[j, pl.ds(0, ch)], out_hbm.at[j, pl.ds(s, ch)]
                )

    @pl.when(wid < _NW - 1)
    def _():
        do_range(wid * _PER, _CH)

    @pl.when(wid == _NW - 1)
    def _():
        do_range((_NW - 1) * _PER, _CH_LAST)


def kernel(values, arousal_bins, dominance_bins, valence_bins):
    vt = values.T
    table = jnp.concatenate([arousal_bins, dominance_bins, valence_bins])
    run = pl.kernel(
        _qbody,
        out_type=jax.ShapeDtypeStruct((3, _N), jnp.int32),
        mesh=plsc.VectorSubcoreMesh(core_axis_name="c", subcore_axis_name="s"),
        compiler_params=pltpu.CompilerParams(
            needs_layout_passes=False, use_tc_tiling_on_sc=False
        ),
        scratch_types=[
            pltpu.VMEM((3 * 256,), jnp.float32),
            pltpu.VMEM((3, _CH), jnp.float32),
            pltpu.VMEM((3, _CH), jnp.int32),
        ],
    )
    out_t = run(vt, table)
    return out_t.T


# speedup vs baseline: 37.9297x; 2.1975x over previous
import jax
import jax.numpy as jnp
from jax import lax
from jax.experimental import pallas as pl
from jax.experimental.pallas import tpu as pltpu
from jax.experimental.pallas import tpu_sc as plsc

_N = 1000000
_NC = 2
_NS = 16
_NW = _NC * _NS
_LANES = 16
_PER = 31264
_PER_LAST = _N - (_NW - 1) * _PER
_NCHUNK = 2
_CH = _PER // _NCHUNK
_CH_LAST = _PER_LAST // _NCHUNK
_STEPS = [128, 64, 32, 16, 8, 4, 2, 1]


def _qbody(vt_hbm, table_hbm, out_hbm, table_v, in_v, out_v):
    wid = lax.axis_index("s") * _NC + lax.axis_index("c")
    pltpu.sync_copy(table_hbm, table_v)
    lane = lax.iota(jnp.int32, _LANES)

    def do_range(base, ch):
        nvec = ch // _LANES
        for c in range(_NCHUNK):
            s = base + c * ch
            for j in range(3):
                pltpu.sync_copy(
                    vt_hbm.at[j, pl.ds(s, ch)], in_v.at[j, pl.ds(0, ch)]
                )

            @plsc.parallel_loop(0, nvec, 1, unroll=4)
            def vbody(g):
                off = g * _LANES
                for j in range(3):
                    x = in_v[j, pl.ds(off, _LANES)]
                    i = (j * 256 + 127) * _LANES + lane
                    for k, st in enumerate(_STEPS):
                        b = plsc.load_gather(table_v, [i])
                        m = b <= x
                        s_next = _STEPS[k + 1] if k + 1 < len(_STEPS) else 1
                        i = i + jnp.where(
                            m, s_next * _LANES, (s_next - st) * _LANES
                        )
                    out_v[j, pl.ds(off, _LANES)] = (i >> 4) & 255

            for j in range(3):
                pltpu.sync_copy(
                    out_v.at[j, pl.ds(0, ch)], out_hbm.at[j, pl.ds(s, ch)]
                )

    @pl.when(wid < _NW - 1)
    def _():
        do_range(wid * _PER, _CH)

    @pl.when(wid == _NW - 1)
    def _():
        do_range((_NW - 1) * _PER, _CH_LAST)


def kernel(values, arousal_bins, dominance_bins, valence_bins):
    vt = values.T
    table = jnp.repeat(
        jnp.concatenate([arousal_bins, dominance_bins, valence_bins]), _LANES
    )
    run = pl.kernel(
        _qbody,
        out_type=jax.ShapeDtypeStruct((3, _N), jnp.int32),
        mesh=plsc.VectorSubcoreMesh(core_axis_name="c", subcore_axis_name="s"),
        compiler_params=pltpu.CompilerParams(
            needs_layout_passes=False, use_tc_tiling_on_sc=False
        ),
        scratch_types=[
            pltpu.VMEM((3 * 256 * _LANES,), jnp.float32),
            pltpu.VMEM((3, _CH), jnp.float32),
            pltpu.VMEM((3, _CH), jnp.int32),
        ],
    )
    out_t = run(vt, table)
    return out_t.T


# speedup vs baseline: 39.0193x; 1.0287x over previous
import jax
import jax.numpy as jnp
from jax import lax
from jax.experimental import pallas as pl
from jax.experimental.pallas import tpu as pltpu
from jax.experimental.pallas import tpu_sc as plsc

_N = 1000000
_NC = 2
_NS = 16
_NW = _NC * _NS
_LANES = 16
_PER = 31296
_PER_LAST = _N - (_NW - 1) * _PER
_NCHUNK = 4
_CH = _PER // _NCHUNK
_CH_LAST = _PER_LAST // _NCHUNK
_STEPS = [128, 64, 32, 16, 8, 4, 2, 1]


def _qbody(
    vt_hbm, table_hbm, out_hbm,
    table_v, in_a, in_b, out_a, out_b,
    sem_ia, sem_ib, sem_oa, sem_ob,
):
    wid = lax.axis_index("s") * _NC + lax.axis_index("c")
    pltpu.sync_copy(table_hbm, table_v)
    lane = lax.iota(jnp.int32, _LANES)

    def do_range(base, ch):
        nvec = ch // _LANES
        ins, outs = [in_a, in_b], [out_a, out_b]
        isems, osems = [sem_ia, sem_ib], [sem_oa, sem_ob]
        in_h, out_h = {}, {}
        for c in range(2):
            in_h[c] = pltpu.async_copy(
                vt_hbm.at[:, pl.ds(base + c * ch, ch)],
                ins[c].at[:, pl.ds(0, ch)],
                isems[c],
            )
        for c in range(_NCHUNK):
            cur = c % 2
            iv, ov = ins[cur], outs[cur]
            in_h[c].wait()
            if c >= 2:
                out_h[c - 2].wait()

            @plsc.parallel_loop(0, nvec, 1, unroll=2)
            def vbody(g):
                off = g * _LANES
                for j in range(3):
                    x = iv[j, pl.ds(off, _LANES)]
                    i = (j * 256 + 127) * _LANES + lane
                    for k, st in enumerate(_STEPS):
                        b = plsc.load_gather(table_v, [i])
                        m = b <= x
                        s_next = _STEPS[k + 1] if k + 1 < len(_STEPS) else 1
                        i = i + jnp.where(
                            m, s_next * _LANES, (s_next - st) * _LANES
                        )
                    ov[j, pl.ds(off, _LANES)] = (i >> 4) & 255

            out_h[c] = pltpu.async_copy(
                ov.at[:, pl.ds(0, ch)],
                out_hbm.at[:, pl.ds(base + c * ch, ch)],
                osems[cur],
            )
            if c + 2 < _NCHUNK:
                in_h[c + 2] = pltpu.async_copy(
                    vt_hbm.at[:, pl.ds(base + (c + 2) * ch, ch)],
                    ins[cur].at[:, pl.ds(0, ch)],
                    isems[cur],
                )
        out_h[_NCHUNK - 2].wait()
        out_h[_NCHUNK - 1].wait()

    @pl.when(wid < _NW - 1)
    def _():
        do_range(wid * _PER, _CH)

    @pl.when(wid == _NW - 1)
    def _():
        do_range((_NW - 1) * _PER, _CH_LAST)


def kernel(values, arousal_bins, dominance_bins, valence_bins):
    vt = values.T
    table = jnp.repeat(
        jnp.concatenate([arousal_bins, dominance_bins, valence_bins]), _LANES
    )
    run = pl.kernel(
        _qbody,
        out_type=jax.ShapeDtypeStruct((3, _N), jnp.int32),
        mesh=plsc.VectorSubcoreMesh(core_axis_name="c", subcore_axis_name="s"),
        compiler_params=pltpu.CompilerParams(
            needs_layout_passes=False, use_tc_tiling_on_sc=False
        ),
        scratch_types=[
            pltpu.VMEM((3 * 256 * _LANES,), jnp.float32),
            pltpu.VMEM((3, _CH), jnp.float32),
            pltpu.VMEM((3, _CH), jnp.float32),
            pltpu.VMEM((3, _CH), jnp.int32),
            pltpu.VMEM((3, _CH), jnp.int32),
            pltpu.SemaphoreType.DMA,
            pltpu.SemaphoreType.DMA,
            pltpu.SemaphoreType.DMA,
            pltpu.SemaphoreType.DMA,
        ],
    )
    out_t = run(vt, table)
    return out_t.T


# speedup vs baseline: 40.4364x; 1.0363x over previous
import jax
import jax.numpy as jnp
from jax import lax
from jax.experimental import pallas as pl
from jax.experimental.pallas import tpu as pltpu
from jax.experimental.pallas import tpu_sc as plsc

_N = 1000000
_NC = 2
_NS = 16
_NW = _NC * _NS
_LANES = 16
_PER = 31296
_NCHUNK = 4
_CH = _PER // _NCHUNK
_STEPS = [128, 64, 32, 16, 8, 4, 2, 1]


def _qbody(
    vt_hbm, table_hbm, out_hbm,
    table_v, in_a, in_b, out_a, out_b,
    sem_ia, sem_ib, sem_oa, sem_ob,
):
    wid = lax.axis_index("s") * _NC + lax.axis_index("c")
    pltpu.sync_copy(table_hbm, table_v)
    lane = lax.iota(jnp.int32, _LANES)
    b0 = [plsc.load_gather(table_v, [(j * 256 + 127) * _LANES + lane])
          for j in range(3)]
    i1_hi = [(j * 256 + 127 + 64) * _LANES + lane for j in range(3)]
    i1_lo = [(j * 256 + 127 - 64) * _LANES + lane for j in range(3)]

    def do_range(base, ch):
        nvec = ch // _LANES
        ins, outs = [in_a, in_b], [out_a, out_b]
        isems, osems = [sem_ia, sem_ib], [sem_oa, sem_ob]
        in_h, out_h = {}, {}
        for c in range(2):
            in_h[c] = pltpu.async_copy(
                vt_hbm.at[:, pl.ds(base + c * ch, ch)],
                ins[c].at[:, pl.ds(0, ch)],
                isems[c],
            )
        for c in range(_NCHUNK):
            cur = c % 2
            iv, ov = ins[cur], outs[cur]
            in_h[c].wait()
            if c >= 2:
                out_h[c - 2].wait()

            @plsc.parallel_loop(0, nvec, 1, unroll=4)
            def vbody(g):
                off = g * _LANES
                for j in range(3):
                    x = iv[j, pl.ds(off, _LANES)]
                    m = b0[j] <= x
                    i = jnp.where(m, i1_hi[j], i1_lo[j])
                    for k, st in enumerate(_STEPS[1:], start=1):
                        b = plsc.load_gather(table_v, [i])
                        m = b <= x
                        s_next = _STEPS[k + 1] if k + 1 < len(_STEPS) else 1
                        i = i + jnp.where(
                            m, s_next * _LANES, (s_next - st) * _LANES
                        )
                    ov[j, pl.ds(off, _LANES)] = (i >> 4) & 255

            out_h[c] = pltpu.async_copy(
                ov.at[:, pl.ds(0, ch)],
                out_hbm.at[:, pl.ds(base + c * ch, ch)],
                osems[cur],
            )
            if c + 2 < _NCHUNK:
                in_h[c + 2] = pltpu.async_copy(
                    vt_hbm.at[:, pl.ds(base + (c + 2) * ch, ch)],
                    ins[cur].at[:, pl.ds(0, ch)],
                    isems[cur],
                )
        out_h[_NCHUNK - 2].wait()
        out_h[_NCHUNK - 1].wait()

    base = jnp.minimum(wid * _PER, _N - _PER)
    do_range(base, _CH)


def kernel(values, arousal_bins, dominance_bins, valence_bins):
    vt = values.T
    table = jnp.repeat(
        jnp.concatenate([arousal_bins, dominance_bins, valence_bins]), _LANES
    )
    run = pl.kernel(
        _qbody,
        out_type=jax.ShapeDtypeStruct((3, _N), jnp.int32),
        mesh=plsc.VectorSubcoreMesh(core_axis_name="c", subcore_axis_name="s"),
        compiler_params=pltpu.CompilerParams(
            needs_layout_passes=False, use_tc_tiling_on_sc=False
        ),
        scratch_types=[
            pltpu.VMEM((3 * 256 * _LANES,), jnp.float32),
            pltpu.VMEM((3, _CH), jnp.float32),
            pltpu.VMEM((3, _CH), jnp.float32),
            pltpu.VMEM((3, _CH), jnp.int32),
            pltpu.VMEM((3, _CH), jnp.int32),
            pltpu.SemaphoreType.DMA,
            pltpu.SemaphoreType.DMA,
            pltpu.SemaphoreType.DMA,
            pltpu.SemaphoreType.DMA,
        ],
    )
    out_t = run(vt, table)
    return out_t.T
